# baseline (speedup 1.0000x reference)
import functools

import jax
import jax.numpy as jnp
from jax import lax
from jax.experimental import pallas as pl
from jax.experimental.pallas import tpu as pltpu
from jax.experimental.pallas import tpu_sc as plsc

_NC = 2
_NS = 16
_NW = _NC * _NS
_CHUNK = 128
_KC = 2
_SLAB = _KC * _CHUNK


@functools.partial(jax.jit, static_argnums=(2, 3))
def _gather_call(y, idx3, n_chunks, d):
    b_per_w = n_chunks * _CHUNK
    n_slabs = n_chunks // _KC
    mesh = plsc.VectorSubcoreMesh(core_axis_name="c", subcore_axis_name="s")

    @functools.partial(
        pl.kernel,
        mesh=mesh,
        out_type=jax.ShapeDtypeStruct((_NW * b_per_w, d), jnp.float32),
        scratch_types=[
            pltpu.VMEM((n_chunks, _CHUNK), jnp.int32),
            pltpu.VMEM((2, _SLAB, d), jnp.float32),
            pltpu.SemaphoreType.DMA,
            pltpu.SemaphoreType.DMA,
        ],
        compiler_params=pltpu.CompilerParams(use_tc_tiling_on_sc=False),
    )
    def body(table_hbm, idx_hbm, out_hbm, idx_v, rows_v, gsem, wsem):
        wid = lax.axis_index("s") * _NC + lax.axis_index("c")
        base = wid * b_per_w
        pltpu.sync_copy(idx_hbm.at[wid], idx_v)

        def out_slab(s):
            return out_hbm.at[pl.ds(base + s * _SLAB, _SLAB)]

        def step(s, carry):
            p = s % 2
            @pl.when(s >= 2)
            def _():
                pltpu.make_async_copy(rows_v.at[p], out_slab(s - 2), wsem).wait()

            for c in range(_KC):
                pltpu.async_copy(
                    table_hbm.at[idx_v.at[s * _KC + c]],
                    rows_v.at[p, pl.ds(c * _CHUNK, _CHUNK)],
                    gsem,
                )
            for c in range(_KC):
                pltpu.make_async_copy(
                    table_hbm.at[idx_v.at[c]],
                    rows_v.at[p, pl.ds(c * _CHUNK, _CHUNK)],
                    gsem,
                ).wait()
            pltpu.async_copy(rows_v.at[p], out_slab(s), wsem)
            return carry

        lax.fori_loop(0, n_slabs, step, 0)
        pltpu.make_async_copy(rows_v.at[(n_slabs - 2) % 2], out_slab(n_slabs - 2), wsem).wait()
        pltpu.make_async_copy(rows_v.at[(n_slabs - 1) % 2], out_slab(n_slabs - 1), wsem).wait()

    return body(y, idx3)


def kernel(x, dim, index):
    v, d = x.shape
    b = index.shape[0]
    idx = index.astype(jnp.int32) + jnp.asarray(dim, jnp.int32)

    y = x.reshape(v // 2, 2 * d)

    grain = _NW * _SLAB
    b_pad = ((b + grain - 1) // grain) * grain
    if b_pad != b:
        idx = jnp.pad(idx, (0, b_pad - b))
    n_chunks = b_pad // (_NW * _CHUNK)
    idx3 = (idx >> 1).reshape(_NW, n_chunks, _CHUNK)

    out128 = _gather_call(y, idx3, n_chunks, 2 * d)
    out128 = out128[:b]
    parity = (idx[:b] & 1).astype(jnp.bool_)[:, None]
    out = jnp.where(parity, out128[:, d:], out128[:, :d])
    return out
